# baseline (speedup 1.0000x reference)
import functools

import jax
import jax.numpy as jnp
from jax import lax
from jax.experimental import pallas as pl
from jax.experimental.pallas import tpu as pltpu
from jax.experimental.pallas import tpu_sc as plsc

_CHUNK = 128
_L = 16


def kernel(x, weight):
    (B,) = x.shape
    V, D = weight.shape

    info = plsc.get_sparse_core_info()
    NC, NS = info.num_cores, info.num_subcores
    NW = NC * NS
    b_per_w = B // NW
    n_ch = b_per_w // _CHUNK
    pack = 128 // D
    Vb = V // pack
    n_grp = (b_per_w * D) // _L

    xi = x.astype(jnp.int32)
    w_blk = weight.reshape(Vb, 128)
    g = (xi // pack).reshape(NW, n_ch, _CHUNK)
    col = ((xi % pack) * D)[:, None] + jnp.arange(D, dtype=jnp.int32)[None, :]
    col = col.reshape(NW, b_per_w * D)

    mesh = plsc.VectorSubcoreMesh(core_axis_name="c", subcore_axis_name="s")

    @functools.partial(
        pl.kernel,
        mesh=mesh,
        out_type=jax.ShapeDtypeStruct((NW, b_per_w * D), jnp.float32),
        scratch_types=[
            pltpu.VMEM((n_ch, _CHUNK), jnp.int32),
            pltpu.VMEM((b_per_w * D,), jnp.int32),
            pltpu.VMEM((b_per_w, 128), jnp.float32),
            pltpu.VMEM((b_per_w * D,), jnp.float32),
            pltpu.SemaphoreType.DMA,
        ],
        compiler_params=pltpu.CompilerParams(needs_layout_passes=False),
    )
    def emb(g_hbm, col_hbm, w_hbm, out_hbm, g_v, col_v, rows_v, out_v, sem):
        wid = lax.axis_index("s") * NC + lax.axis_index("c")
        pltpu.sync_copy(g_hbm.at[wid], g_v)
        pltpu.sync_copy(col_hbm.at[wid], col_v)
        copies = []
        for j in range(n_ch):
            c = pltpu.make_async_copy(
                w_hbm.at[g_v.at[j]],
                rows_v.at[pl.ds(j * _CHUNK, _CHUNK)],
                sem,
            )
            c.start()
            copies.append(c)
        for c in copies:
            c.wait()

        def body(k, _):
            row16 = jnp.full((_L,), k // (D // _L), dtype=jnp.int32)
            col16 = col_v[pl.ds(k * _L, _L)]
            v = plsc.load_gather(rows_v, [row16, col16])
            out_v[pl.ds(k * _L, _L)] = v
            return 0

        lax.fori_loop(0, n_grp, body, 0)
        pltpu.sync_copy(out_v, out_hbm.at[wid])

    return emb(g, col, w_blk).reshape(B, D)


# speedup vs baseline: 1.0328x; 1.0328x over previous
import functools

import jax
import jax.numpy as jnp
from jax import lax
from jax.experimental import pallas as pl
from jax.experimental.pallas import tpu as pltpu
from jax.experimental.pallas import tpu_sc as plsc

_CHUNK = 128


def kernel(x, weight):
    (B,) = x.shape
    V, D = weight.shape

    info = plsc.get_sparse_core_info()
    NC, NS = info.num_cores, info.num_subcores
    NW = NC * NS
    b_per_w = B // NW
    n_ch = b_per_w // _CHUNK

    x_shaped = x.astype(jnp.int32).reshape(NW, n_ch, _CHUNK)

    mesh = plsc.VectorSubcoreMesh(core_axis_name="c", subcore_axis_name="s")

    @functools.partial(
        pl.kernel,
        mesh=mesh,
        out_type=jax.ShapeDtypeStruct((B, D), jnp.float32),
        scratch_types=[
            pltpu.VMEM((n_ch, _CHUNK), jnp.int32),
            pltpu.VMEM((b_per_w, D), jnp.float32),
            pltpu.SemaphoreType.DMA,
        ],
        compiler_params=pltpu.CompilerParams(
            use_tc_tiling_on_sc=False,
            needs_layout_passes=False,
        ),
    )
    def emb(x_hbm, w_hbm, out_hbm, idx_v, rows_v, sem):
        wid = lax.axis_index("s") * NC + lax.axis_index("c")
        base = wid * b_per_w
        pltpu.sync_copy(x_hbm.at[wid], idx_v)
        copies = []
        for j in range(n_ch):
            c = pltpu.make_async_copy(
                w_hbm.at[idx_v.at[j]],
                rows_v.at[pl.ds(j * _CHUNK, _CHUNK)],
                sem,
            )
            c.start()
            copies.append(c)
        for c in copies:
            c.wait()
        pltpu.sync_copy(rows_v, out_hbm.at[pl.ds(base, b_per_w)])

    return emb(x_shaped, weight)


# speedup vs baseline: 1.6169x; 1.5656x over previous
import functools

import jax
import jax.numpy as jnp
from jax import lax
from jax.experimental import pallas as pl
from jax.experimental.pallas import tpu as pltpu
from jax.experimental.pallas import tpu_sc as plsc

_WAVE = 16


def kernel(x, weight):
    (B,) = x.shape
    V, D = weight.shape

    info = plsc.get_sparse_core_info()
    NC, NS = info.num_cores, info.num_subcores
    NW = NC * NS
    b_per_w = B // NW
    n_wave = b_per_w // _WAVE

    xi = x.astype(jnp.int32)

    mesh = plsc.VectorSubcoreMesh(core_axis_name="c", subcore_axis_name="s")

    @functools.partial(
        pl.kernel,
        mesh=mesh,
        out_type=jax.ShapeDtypeStruct((B, D), jnp.float32),
        scratch_types=[
            pltpu.VMEM((b_per_w,), jnp.int32),
            pltpu.VMEM((b_per_w, D), jnp.float32),
            pltpu.SemaphoreType.DMA,
        ],
        compiler_params=pltpu.CompilerParams(needs_layout_passes=False),
    )
    def emb(x_hbm, w_hbm, out_hbm, x_v, rows_v, sem):
        wid = lax.axis_index("s") * NC + lax.axis_index("c")
        base = wid * b_per_w
        pltpu.sync_copy(x_hbm.at[pl.ds(base, b_per_w)], x_v)

        def wave(wv, _):
            xv = x_v[pl.ds(wv * _WAVE, _WAVE)]
            copies = []
            for i in range(_WAVE):
                e = wv * _WAVE + i
                t = xv[i]
                c = pltpu.make_async_copy(
                    w_hbm.at[t], rows_v.at[e], sem
                )
                c.start()
                copies.append(c)
            for c in copies:
                c.wait()
            return 0

        lax.fori_loop(0, n_wave, wave, 0)
        pltpu.sync_copy(rows_v, out_hbm.at[pl.ds(base, b_per_w)])

    return emb(xi, weight)
